# baseline (speedup 1.0000x reference)
import functools

import jax
import jax.numpy as jnp
from jax.experimental import pallas as pl
from jax.experimental.pallas import tpu as pltpu

_WP = 32
_CHUNK = 128


def _block_body(x_ref, we_ref, wd_ref, bd_ref, wp_ref, bp_ref,
                o_ref, pf0, pf1, pf2, hdw,
                *, H, W, C, hid, Sp):
    xin = x_ref[0]

    h = jax.lax.dot_general(
        xin, we_ref[...],
        (((0,), (0,)), ((), ())),
        preferred_element_type=jnp.float32)
    hb = jnp.clip(h, 0.0, 6.0).astype(jnp.bfloat16)

    zb = jnp.zeros((32, hid), jnp.bfloat16)
    zr = jnp.zeros((1, hid), jnp.bfloat16)
    for pf in (pf0, pf1, pf2):
        pf[pl.ds(0, 32)] = zb
        pf[pl.ds(Sp + 32, 32)] = zb
    for r in range(1, H + 1):
        pf0[pl.ds(_WP * r, 1)] = zr
        pf2[pl.ds(_WP * r + W - 1, 1)] = zr
    for r in range(1, H + 1):
        piece = hb[(r - 1) * W:r * W]
        pf0[pl.ds(_WP * r + 1, W)] = piece
        pf1[pl.ds(_WP * r, W)] = piece
        pf2[pl.ds(_WP * r - 1, W)] = piece

    pfs = (pf0, pf1, pf2)
    for ci in range(Sp // _CHUNK):
        base = ci * _CHUNK
        acc = None
        for dh in range(3):
            for dx in range(3):
                slab = pfs[dx][pl.ds(_WP * dh + base, _CHUNK)]
                term = slab * wd_ref[3 * dh + dx, :]
                acc = term if acc is None else acc + term
        hdw[pl.ds(base, _CHUNK)] = jnp.clip(acc + bd_ref[...], 0.0, 6.0)

    y = jax.lax.dot_general(
        wp_ref[...], hdw[...],
        (((0,), (1,)), ((), ())),
        preferred_element_type=jnp.float32)
    res = xin[0:C].astype(jnp.float32)
    bp = bp_ref[...]
    for hrow in range(H):
        piece = y[:, _WP * hrow:_WP * hrow + W] + bp + res[:, W * hrow:W * hrow + W]
        o_ref[0, :, pl.ds(W * hrow, W)] = piece.astype(jnp.bfloat16)


def kernel(x_nchw, w_exp, b_exp, w_dw, b_dw, w_proj, b_proj):
    N, C, H, W = x_nchw.shape
    S = H * W
    hid = w_exp.shape[1]
    Sp = H * _WP

    xc = jnp.concatenate(
        [x_nchw.reshape(N, C, S), jnp.ones((N, 1, S), x_nchw.dtype)],
        axis=1).astype(jnp.bfloat16)

    we = jnp.concatenate([w_exp, b_exp], axis=0).astype(jnp.bfloat16)
    wd = w_dw.astype(jnp.bfloat16)
    bd = b_dw.astype(jnp.bfloat16)
    wp = w_proj.astype(jnp.bfloat16)
    bp = jnp.transpose(b_proj)

    body = functools.partial(_block_body, H=H, W=W, C=C, hid=hid, Sp=Sp)

    def full(shape):
        nd = len(shape)
        return pl.BlockSpec(shape, lambda n, nd=nd: (0,) * nd)

    out = pl.pallas_call(
        body,
        out_shape=jax.ShapeDtypeStruct((N, C, S), jnp.bfloat16),
        grid=(N,),
        in_specs=[
            pl.BlockSpec((1, C + 1, S), lambda n: (n, 0, 0)),
            full(we.shape),
            full(wd.shape),
            full(bd.shape),
            full(wp.shape),
            full(bp.shape),
        ],
        out_specs=pl.BlockSpec((1, C, S), lambda n: (n, 0, 0)),
        scratch_shapes=[
            pltpu.VMEM((Sp + 64, hid), jnp.bfloat16),
            pltpu.VMEM((Sp + 64, hid), jnp.bfloat16),
            pltpu.VMEM((Sp + 64, hid), jnp.bfloat16),
            pltpu.VMEM((Sp, hid), jnp.bfloat16),
        ],
        compiler_params=pltpu.CompilerParams(
            dimension_semantics=("parallel",)),
    )(xc, we, wd, bd, wp, bp)

    return out.reshape(N, C, H, W).astype(jnp.float32)


# speedup vs baseline: 1.1143x; 1.1143x over previous
import functools

import jax
import jax.numpy as jnp
from jax.experimental import pallas as pl
from jax.experimental.pallas import tpu as pltpu

_WP = 32
_CHUNK = 128
_PAIR = 2


def _one_image(xa, we_ref, wd_ref, bd_ref, wp_ref, bp_ref,
               pf0, pf1, pf2, hdw, *, C, hid, Sp):
    h = jax.lax.dot_general(
        xa, we_ref[...],
        (((0,), (0,)), ((), ())),
        preferred_element_type=jnp.float32)
    hb = jnp.clip(h, 0.0, 6.0).astype(jnp.bfloat16)

    zb = jnp.zeros((40, hid), jnp.bfloat16)
    pf0[pl.ds(0, 40)] = zb
    pf1[pl.ds(0, 32)] = zb[:32]
    pf2[pl.ds(0, 32)] = zb[:32]
    pf0[pl.ds(Sp + 32, 32)] = zb[:32]
    pf1[pl.ds(Sp + 32, 32)] = zb[:32]
    pf2[pl.ds(Sp + 32, 32)] = zb[:32]
    pf0[pl.ds(33, Sp)] = hb
    pf1[pl.ds(32, Sp)] = hb
    pf2[pl.ds(31, Sp)] = hb

    pfs = (pf0, pf1, pf2)
    for ci in range(Sp // _CHUNK):
        base = ci * _CHUNK
        acc = None
        for dh in range(3):
            for dx in range(3):
                slab = pfs[dx][pl.ds(32 * dh + base, _CHUNK)]
                term = slab * wd_ref[3 * dh + dx, :]
                acc = term if acc is None else acc + term
        hdw[pl.ds(base, _CHUNK)] = jnp.clip(acc + bd_ref[...], 0.0, 6.0)

    y = jax.lax.dot_general(
        wp_ref[...], hdw[...],
        (((0,), (1,)), ((), ())),
        preferred_element_type=jnp.float32)
    return y + bp_ref[...] + xa[0:C].astype(jnp.float32)


def _block_body(x_ref, we_ref, wd_ref, bd_ref, wp_ref, bp_ref, o_ref,
                pf0a, pf1a, pf2a, hdwa, pf0b, pf1b, pf2b, hdwb,
                *, C, hid, Sp):
    args = (we_ref, wd_ref, bd_ref, wp_ref, bp_ref)
    kw = dict(C=C, hid=hid, Sp=Sp)
    o_ref[0] = _one_image(x_ref[0], *args, pf0a, pf1a, pf2a, hdwa, **kw)
    o_ref[1] = _one_image(x_ref[1], *args, pf0b, pf1b, pf2b, hdwb, **kw)


def kernel(x_nchw, w_exp, b_exp, w_dw, b_dw, w_proj, b_proj):
    N, C, H, W = x_nchw.shape
    hid = w_exp.shape[1]
    Sp = H * _WP

    xb = x_nchw.astype(jnp.bfloat16)
    xp = jnp.pad(xb, ((0, 0), (0, 0), (0, 0), (0, _WP - W)))
    xr = xp.reshape(N, C, Sp)
    mask = (jax.lax.iota(jnp.int32, Sp) % _WP < W).astype(jnp.bfloat16)
    xa = jnp.concatenate(
        [xr, jnp.broadcast_to(mask, (N, 1, Sp))], axis=1)

    we = jnp.concatenate([w_exp, b_exp], axis=0).astype(jnp.bfloat16)
    wd = w_dw.astype(jnp.bfloat16)
    bd = b_dw.astype(jnp.bfloat16)
    wp = w_proj.astype(jnp.bfloat16)
    bp = jnp.transpose(b_proj)

    body = functools.partial(_block_body, C=C, hid=hid, Sp=Sp)

    def full(shape):
        nd = len(shape)
        return pl.BlockSpec(shape, lambda n, nd=nd: (0,) * nd)

    pf = lambda: pltpu.VMEM((Sp + 64, hid), jnp.bfloat16)
    out = pl.pallas_call(
        body,
        out_shape=jax.ShapeDtypeStruct((N, C, Sp), jnp.float32),
        grid=(N // _PAIR,),
        in_specs=[
            pl.BlockSpec((_PAIR, C + 1, Sp), lambda n: (n, 0, 0)),
            full(we.shape),
            full(wd.shape),
            full(bd.shape),
            full(wp.shape),
            full(bp.shape),
        ],
        out_specs=pl.BlockSpec((_PAIR, C, Sp), lambda n: (n, 0, 0)),
        scratch_shapes=[
            pf(), pf(), pf(), pltpu.VMEM((Sp, hid), jnp.bfloat16),
            pf(), pf(), pf(), pltpu.VMEM((Sp, hid), jnp.bfloat16),
        ],
        compiler_params=pltpu.CompilerParams(
            dimension_semantics=("parallel",)),
    )(xa, we, wd, bd, wp, bp)

    return out.reshape(N, C, H, _WP)[..., :W]


# speedup vs baseline: 1.1810x; 1.0598x over previous
import functools

import jax
import jax.numpy as jnp
from jax.experimental import pallas as pl
from jax.experimental.pallas import tpu as pltpu

_WP = 32
_CHUNK = 128
_PAIR = 2


def _one_image(xa, we_ref, wd_ref, bd_ref, wp_ref, bp_ref,
               pf0, pf1, pf2, hdw, *, C, W, hid, Sp):
    zb = jnp.zeros((40, hid), jnp.bfloat16)
    pf0[pl.ds(0, 40)] = zb
    pf1[pl.ds(0, 32)] = zb[:32]
    pf2[pl.ds(0, 32)] = zb[:32]
    pf0[pl.ds(Sp + 32, 32)] = zb[:32]
    pf1[pl.ds(Sp + 32, 32)] = zb[:32]
    pf2[pl.ds(Sp + 32, 32)] = zb[:32]

    mrow = (jax.lax.broadcasted_iota(jnp.int32, (1, _CHUNK), 1) % _WP
            < W).astype(jnp.bfloat16)

    for ci in range(Sp // _CHUNK):
        base = ci * _CHUNK
        xc = jnp.concatenate([xa[:, base:base + _CHUNK], mrow], axis=0)
        h = jax.lax.dot_general(
            xc, we_ref[...],
            (((0,), (0,)), ((), ())),
            preferred_element_type=jnp.float32)
        hbc = jnp.clip(h, 0.0, 6.0).astype(jnp.bfloat16)
        pf0[pl.ds(33 + base, _CHUNK)] = hbc
        pf1[pl.ds(32 + base, _CHUNK)] = hbc
        pf2[pl.ds(31 + base, _CHUNK)] = hbc

    pfs = (pf0, pf1, pf2)
    for ci in range(Sp // _CHUNK):
        base = ci * _CHUNK
        acc = None
        for dh in range(3):
            for dx in range(3):
                slab = pfs[dx][pl.ds(32 * dh + base, _CHUNK)]
                term = slab * wd_ref[3 * dh + dx, :]
                acc = term if acc is None else acc + term
        hdw[pl.ds(base, _CHUNK)] = jnp.clip(acc + bd_ref[...], 0.0, 6.0)

    y = jax.lax.dot_general(
        wp_ref[...], hdw[...],
        (((0,), (1,)), ((), ())),
        preferred_element_type=jnp.float32)
    y = y + bp_ref[...] + xa.astype(jnp.float32)
    return y.astype(jnp.bfloat16)


def _block_body(x_ref, we_ref, wd_ref, bd_ref, wp_ref, bp_ref, o_ref,
                pf0a, pf1a, pf2a, hdwa, pf0b, pf1b, pf2b, hdwb,
                *, C, W, hid, Sp):
    args = (we_ref, wd_ref, bd_ref, wp_ref, bp_ref)
    kw = dict(C=C, W=W, hid=hid, Sp=Sp)
    o_ref[0] = _one_image(x_ref[0], *args, pf0a, pf1a, pf2a, hdwa, **kw)
    o_ref[1] = _one_image(x_ref[1], *args, pf0b, pf1b, pf2b, hdwb, **kw)


def kernel(x_nchw, w_exp, b_exp, w_dw, b_dw, w_proj, b_proj):
    N, C, H, W = x_nchw.shape
    hid = w_exp.shape[1]
    Sp = H * _WP

    xb = x_nchw.astype(jnp.bfloat16)
    xp = jnp.pad(xb, ((0, 0), (0, 0), (0, 0), (0, _WP - W)))
    xa = xp.reshape(N, C, Sp)

    we = jnp.concatenate([w_exp, b_exp], axis=0).astype(jnp.bfloat16)
    wd = w_dw.astype(jnp.bfloat16)
    bd = b_dw.astype(jnp.bfloat16)
    wp = w_proj.astype(jnp.bfloat16)
    bp = jnp.transpose(b_proj)

    body = functools.partial(_block_body, C=C, W=W, hid=hid, Sp=Sp)

    def full(shape):
        nd = len(shape)
        return pl.BlockSpec(shape, lambda n, nd=nd: (0,) * nd)

    pf = lambda: pltpu.VMEM((Sp + 64, hid), jnp.bfloat16)
    out = pl.pallas_call(
        body,
        out_shape=jax.ShapeDtypeStruct((N, C, Sp), jnp.bfloat16),
        grid=(N // _PAIR,),
        in_specs=[
            pl.BlockSpec((_PAIR, C, Sp), lambda n: (n, 0, 0)),
            full(we.shape),
            full(wd.shape),
            full(bd.shape),
            full(wp.shape),
            full(bp.shape),
        ],
        out_specs=pl.BlockSpec((_PAIR, C, Sp), lambda n: (n, 0, 0)),
        scratch_shapes=[
            pf(), pf(), pf(), pltpu.VMEM((Sp, hid), jnp.bfloat16),
            pf(), pf(), pf(), pltpu.VMEM((Sp, hid), jnp.bfloat16),
        ],
        compiler_params=pltpu.CompilerParams(
            dimension_semantics=("parallel",)),
    )(xa, we, wd, bd, wp, bp)

    return out.reshape(N, C, H, _WP)[..., :W].astype(jnp.float32)
